# Initial kernel scaffold; baseline (speedup 1.0000x reference)
import jax
import jax.numpy as jnp
from jax.experimental import pallas as pl


def kernel(tokens, W_pos):
    raise NotImplementedError("write your pallas kernel here")


# speedup vs baseline: 1.0098x; 1.0098x over previous
import jax
import jax.numpy as jnp
from jax.experimental import pallas as pl


def _bcast_kernel(w_ref, out_ref):
    out_ref[...] = jnp.broadcast_to(w_ref[...][None, :, :], out_ref.shape)


def kernel(tokens, W_pos):
    batch, seq_len = tokens.shape
    d = W_pos.shape[1]
    bs = 512
    grid = (seq_len // bs,)
    out = pl.pallas_call(
        _bcast_kernel,
        grid=grid,
        in_specs=[pl.BlockSpec((bs, d), lambda i: (i, 0))],
        out_specs=pl.BlockSpec((batch, bs, d), lambda i: (0, i, 0)),
        out_shape=jax.ShapeDtypeStruct((batch, seq_len, d), W_pos.dtype),
    )(W_pos[:seq_len])
    return out


# speedup vs baseline: 1.0355x; 1.0254x over previous
import jax
import jax.numpy as jnp
from jax.experimental import pallas as pl


def _bcast_kernel(w_ref, out_ref):
    out_ref[...] = jnp.broadcast_to(w_ref[...][None, :, :], out_ref.shape)


def kernel(tokens, W_pos):
    batch, seq_len = tokens.shape
    d = W_pos.shape[1]
    bs = 1024
    grid = (seq_len // bs,)
    out = pl.pallas_call(
        _bcast_kernel,
        grid=grid,
        in_specs=[pl.BlockSpec((bs, d), lambda i: (i, 0))],
        out_specs=pl.BlockSpec((batch, bs, d), lambda i: (0, i, 0)),
        out_shape=jax.ShapeDtypeStruct((batch, seq_len, d), W_pos.dtype),
    )(W_pos[:seq_len])
    return out
